# baseline (speedup 1.0000x reference)
import functools

import jax
import jax.numpy as jnp
from jax import lax
from jax.experimental import pallas as pl
from jax.experimental.pallas import tpu as pltpu
from jax.experimental.pallas import tpu_sc as plsc

SC_B = 3


def _sc_gather(xf, perm_i32, n_b, S, D):
    info = plsc.get_sparse_core_info()
    NC, NS, L = info.num_cores, info.num_subcores, info.num_lanes
    NW = NC * NS
    seg = S // NW
    rows_per_w = n_b * seg
    CH = 16
    NBUF = 6
    n_cs = seg // CH
    n_ch = rows_per_w // CH

    mesh = plsc.VectorSubcoreMesh(core_axis_name="c", subcore_axis_name="s")

    @functools.partial(
        pl.kernel,
        mesh=mesh,
        out_type=jax.ShapeDtypeStruct((n_b * S, D), jnp.float32),
        scratch_types=[
            pltpu.VMEM((rows_per_w,), jnp.int32),
        ] + [pltpu.VMEM((CH, D), jnp.float32) for _ in range(NBUF)] + [
            pltpu.SemaphoreType.DMA,
            pltpu.SemaphoreType.DMA,
        ],
    )
    def _k(x_hbm, perm_hbm, out_hbm, idx_v, *bufs_and_sems):
        bufs = bufs_and_sems[:NBUF]
        sem_g, sem_s = bufs_and_sems[NBUF:]
        wid = lax.axis_index("s") * NC + lax.axis_index("c")
        seq0 = pl.multiple_of(wid * seg, CH)

        pltpu.sync_copy(perm_hbm.at[pl.ds(seq0, seg)], idx_v.at[pl.ds(0, seg)])
        for b in range(n_b - 1, -1, -1):
            boff = b * S
            for j in range(seg // L):
                sl_src = pl.ds(j * L, L)
                sl_dst = pl.ds(b * seg + j * L, L)
                idx_v[sl_dst] = idx_v[sl_src] + boff

        def out_base(c):
            b = c // n_cs
            return pl.multiple_of(b * S + wid * seg + (c % n_cs) * CH, CH)

        def gather_start(c):
            return pltpu.async_copy(
                x_hbm.at[idx_v.at[pl.ds(c * CH, CH)]], bufs[c % NBUF], sem_g
            )

        def scatter_start(c):
            return pltpu.async_copy(
                bufs[c % NBUF], out_hbm.at[pl.ds(out_base(c), CH)], sem_s
            )

        P = NBUF - 2
        g = [None] * n_ch
        s = [None] * n_ch
        for c in range(min(P, n_ch)):
            g[c] = gather_start(c)
        for c in range(n_ch):
            g[c].wait()
            s[c] = scatter_start(c)
            nxt = c + P
            if nxt < n_ch:
                prev = nxt - NBUF
                if prev >= 0:
                    s[prev].wait()
                g[nxt] = gather_start(nxt)
        for c in range(max(0, n_ch - NBUF), n_ch):
            s[c].wait()

    return _k(xf, perm_i32)


def kernel(x, perm):
    B, S, D = x.shape
    perm_i32 = perm.astype(jnp.int32)
    xf = x.reshape(B * S, D)
    out_sc = _sc_gather(xf, perm_i32, SC_B, S, D).reshape(SC_B, S, D)
    if SC_B < B:
        out_tc = jnp.take(x[SC_B:], perm_i32, axis=1)
        return jnp.concatenate([out_sc, out_tc], axis=0)
    return out_sc
